# baseline (speedup 1.0000x reference)
import jax
import jax.numpy as jnp
from jax import lax
from jax.experimental import pallas as pl
from jax.experimental.pallas import tpu as pltpu
from jax.experimental.pallas import tpu_sc as plsc

_B, _P, _D = 64, 576, 768
_NC, _NS = 2, 16
_NW = _NC * _NS
_CH = 32
_NK = _P // _CH
_NQ = 2 * _NK
_LANES = 16


def _sc_body(patches_hbm, table_hbm, out_hbm, tbuf, buf0, buf1,
             isem0, isem1, osem0, osem1):
    c = lax.axis_index("c")
    s = lax.axis_index("s")
    w = s * _NC + c
    b0 = 2 * w

    bufs = (buf0, buf1)
    isems = (isem0, isem1)
    osems = (osem0, osem1)

    def base(q):
        return pl.multiple_of((b0 + q % 2) * _P + (q // 2) * _CH, _CH)

    def in_copy(q, j):
        return pltpu.make_async_copy(
            patches_hbm.at[pl.ds(base(q), _CH)], bufs[j], isems[j])

    def out_copy(q, j):
        return pltpu.make_async_copy(
            bufs[j], out_hbm.at[pl.ds(base(q), _CH)], osems[j])

    def add_chunk(j):
        buf = bufs[j]

        def row(r, carry):
            for g in range(_D // _LANES):
                sl = pl.ds(g * _LANES, _LANES)
                buf[r, sl] = buf[r, sl] + tbuf[r, sl]
            return carry
        lax.fori_loop(0, _CH, row, 0)

    def chunk(q, j, first=False, last=False):
        jn = 1 - j
        if j == 0:
            pltpu.sync_copy(table_hbm.at[pl.ds(pl.multiple_of((q // 2) * _CH, _CH), _CH)], tbuf)
        in_copy(q, j).wait()
        if not first:
            out_copy(q - 1, jn).wait()
        if not last:
            in_copy(q + 1, jn).start()
        add_chunk(j)
        out_copy(q, j).start()

    in_copy(0, 0).start()
    chunk(0, 0, first=True)
    chunk(1, 1)

    def pair(i, carry):
        chunk(2 * i, 0)
        chunk(2 * i + 1, 1)
        return carry

    lax.fori_loop(1, _NQ // 2 - 1, pair, 0)

    chunk(_NQ - 2, 0)
    chunk(_NQ - 1, 1, last=True)
    out_copy(_NQ - 1, (_NQ - 1) % 2).wait()


def kernel(encoded_patches, position_table):
    B, P, D = encoded_patches.shape
    rows = encoded_patches.reshape(B * P, D)
    mesh = plsc.VectorSubcoreMesh(core_axis_name="c", subcore_axis_name="s")
    out = pl.kernel(
        _sc_body,
        out_type=jax.ShapeDtypeStruct((B * P, D), encoded_patches.dtype),
        mesh=mesh,
        scratch_types=[
            pltpu.VMEM((_CH, _D), jnp.float32),
            pltpu.VMEM((_CH, _D), jnp.float32),
            pltpu.VMEM((_CH, _D), jnp.float32),
            pltpu.SemaphoreType.DMA,
            pltpu.SemaphoreType.DMA,
            pltpu.SemaphoreType.DMA,
            pltpu.SemaphoreType.DMA,
        ],
    )(rows, position_table)
    return out.reshape(B, P, D)


# speedup vs baseline: 2.0326x; 2.0326x over previous
import jax
import jax.numpy as jnp
from jax import lax
from jax.experimental import pallas as pl
from jax.experimental.pallas import tpu as pltpu

_B, _P, _D = 64, 576, 768
_NSLOT = 4
_LOOK = 2


def _pipe_body(patches_hbm, table_hbm, out_hbm, table_v, bufs, isems, osems, tsem):
    pltpu.make_async_copy(table_hbm, table_v, tsem).start()

    def in_copy(i, slot):
        return pltpu.make_async_copy(
            patches_hbm.at[pl.ds(i, 1)], bufs.at[slot], isems.at[slot])

    def out_copy(i, slot):
        return pltpu.make_async_copy(
            bufs.at[slot], out_hbm.at[pl.ds(i, 1)], osems.at[slot])

    in_copy(0, 0).start()
    in_copy(1, 1).start()
    pltpu.make_async_copy(table_hbm, table_v, tsem).wait()

    def do(i, slot, prefetch, waitout):
        in_copy(i, slot).wait()
        bufs[slot, 0] = bufs[slot, 0] + table_v[...]
        out_copy(i, slot).start()
        nslot = (i + _LOOK) % _NSLOT
        if prefetch:
            if waitout:
                out_copy(i + _LOOK - _NSLOT, nslot).wait()
            in_copy(i + _LOOK, nslot).start()

    do(0, 0, True, False)
    do(1, 1, True, False)

    def group(g, carry):
        i0 = _NSLOT * g + 2
        for k in range(_NSLOT):
            do(i0 + k, (2 + k) % _NSLOT, True, True)
        return carry

    lax.fori_loop(0, (_B - 2 - _NSLOT) // _NSLOT, group, 0)

    do(_B - 6, (_B - 6) % _NSLOT, True, True)
    do(_B - 5, (_B - 5) % _NSLOT, True, True)
    do(_B - 4, (_B - 4) % _NSLOT, True, True)
    do(_B - 3, (_B - 3) % _NSLOT, True, True)
    do(_B - 2, (_B - 2) % _NSLOT, False, False)
    do(_B - 1, (_B - 1) % _NSLOT, False, False)
    for i in range(_B - _NSLOT, _B):
        out_copy(i, i % _NSLOT).wait()


def kernel(encoded_patches, position_table):
    B, P, D = encoded_patches.shape
    return pl.pallas_call(
        _pipe_body,
        in_specs=[
            pl.BlockSpec(memory_space=pl.ANY),
            pl.BlockSpec(memory_space=pl.ANY),
        ],
        out_specs=pl.BlockSpec(memory_space=pl.ANY),
        out_shape=jax.ShapeDtypeStruct((B, P, D), encoded_patches.dtype),
        scratch_shapes=[
            pltpu.VMEM((P, D), jnp.float32),
            pltpu.VMEM((_NSLOT, 1, P, D), jnp.float32),
            pltpu.SemaphoreType.DMA((_NSLOT,)),
            pltpu.SemaphoreType.DMA((_NSLOT,)),
            pltpu.SemaphoreType.DMA,
        ],
    )(encoded_patches, position_table)


# speedup vs baseline: 2.3679x; 1.1650x over previous
import jax
import jax.numpy as jnp
from jax.experimental import pallas as pl
from jax.experimental.pallas import tpu as pltpu

_B, _P, _D = 64, 576, 768
_CB = 4
_NCH = _B // _CB
_NSLOT = 4
_LOOK = 2


def _pipe_body(patches_hbm, table_hbm, out_hbm, table_v, bufs, isems, osems, tsem):
    pltpu.make_async_copy(table_hbm, table_v, tsem).start()

    def in_copy(i, slot):
        return pltpu.make_async_copy(
            patches_hbm.at[pl.ds(i * _CB, _CB)], bufs.at[slot], isems.at[slot])

    def out_copy(i, slot):
        return pltpu.make_async_copy(
            bufs.at[slot], out_hbm.at[pl.ds(i * _CB, _CB)], osems.at[slot])

    for i in range(_LOOK):
        in_copy(i, i % _NSLOT).start()
    pltpu.make_async_copy(table_hbm, table_v, tsem).wait()

    for i in range(_NCH):
        slot = i % _NSLOT
        in_copy(i, slot).wait()
        for b in range(_CB):
            bufs[slot, b] = bufs[slot, b] + table_v[...]
        out_copy(i, slot).start()
        if i + _LOOK < _NCH:
            nslot = (i + _LOOK) % _NSLOT
            if i + _LOOK >= _NSLOT:
                out_copy(i + _LOOK - _NSLOT, nslot).wait()
            in_copy(i + _LOOK, nslot).start()

    for i in range(_NCH - _NSLOT, _NCH):
        out_copy(i, i % _NSLOT).wait()


def kernel(encoded_patches, position_table):
    B, P, D = encoded_patches.shape
    return pl.pallas_call(
        _pipe_body,
        in_specs=[
            pl.BlockSpec(memory_space=pl.ANY),
            pl.BlockSpec(memory_space=pl.ANY),
        ],
        out_specs=pl.BlockSpec(memory_space=pl.ANY),
        out_shape=jax.ShapeDtypeStruct((B, P, D), encoded_patches.dtype),
        scratch_shapes=[
            pltpu.VMEM((P, D), jnp.float32),
            pltpu.VMEM((_NSLOT, _CB, P, D), jnp.float32),
            pltpu.SemaphoreType.DMA((_NSLOT,)),
            pltpu.SemaphoreType.DMA((_NSLOT,)),
            pltpu.SemaphoreType.DMA,
        ],
    )(encoded_patches, position_table)


# speedup vs baseline: 2.3698x; 1.0008x over previous
import jax
import jax.numpy as jnp
from jax.experimental import pallas as pl
from jax.experimental.pallas import tpu as pltpu


def _add_kernel(p_ref, t_ref, o_ref):
    o_ref[...] = p_ref[...] + t_ref[...]


def kernel(encoded_patches, position_table):
    B, P, D = encoded_patches.shape
    BB = 8
    return pl.pallas_call(
        _add_kernel,
        grid=(B // BB,),
        in_specs=[
            pl.BlockSpec((BB, P, D), lambda i: (i, 0, 0)),
            pl.BlockSpec((P, D), lambda i: (0, 0)),
        ],
        out_specs=pl.BlockSpec((BB, P, D), lambda i: (i, 0, 0)),
        out_shape=jax.ShapeDtypeStruct((B, P, D), encoded_patches.dtype),
        compiler_params=pltpu.CompilerParams(vmem_limit_bytes=128 * 1024 * 1024),
    )(encoded_patches, position_table)
